# baseline (speedup 1.0000x reference)
import functools

import jax
import jax.numpy as jnp
import numpy as np
from jax.experimental import pallas as pl
from jax.experimental.pallas import tpu as pltpu

_BQ = 512
_BK = 1024
_G = 4
_NEG = -1e30


def _flash_body(kmin_ref, smax_ref, cu_ref, q_ref, k_ref, v_ref, o_ref,
                s_ref, acc_ref, *, num_segs, g, bq, bk):
    hg = pl.program_id(0)
    del hg
    i = pl.program_id(1)
    d = q_ref.shape[-1]

    kmin = kmin_ref[i]
    smax = smax_ref[i]
    jmax = (i * bq + bq - 1) // bk

    ones_bk = jnp.ones((bk, 8), jnp.bfloat16)

    def body(jb, carry):
        ms, ls = carry
        for gg in range(g):
            s_ref[gg] = jax.lax.dot_general(
                k_ref[gg, pl.ds(jb * bk, bk), :], q_ref[gg],
                (((1,), (1,)), ((), ())),
                preferred_element_type=jnp.float32)

        @pl.when(jb * bk < smax)
        def _segmask():
            qpos = i * bq + jax.lax.broadcasted_iota(jnp.int32, (1, bq), 1)
            seg_start = jnp.zeros((1, bq), jnp.int32)
            for b in range(1, num_segs + 1):
                c = cu_ref[b]
                seg_start = jnp.where(qpos >= c, c, seg_start)
            kpos = jb * bk + jax.lax.broadcasted_iota(jnp.int32, (bk, 1), 0)
            sel = kpos >= seg_start
            for gg in range(g):
                s_ref[gg] = jnp.where(sel, s_ref[gg], _NEG)

        def update(gg, s, m_prev, l_prev):
            m_cur = jnp.max(s, axis=0, keepdims=True)
            m_new = jnp.maximum(m_prev, m_cur)
            alpha = jnp.exp2(m_prev - m_new)
            p = jnp.exp2(s - m_new).astype(jnp.bfloat16)
            lsum = jax.lax.dot_general(
                ones_bk, p, (((0,), (0,)), ((), ())),
                preferred_element_type=jnp.float32)
            l_new = l_prev * alpha + lsum[0:1, :]
            pv = jax.lax.dot_general(
                v_ref[gg, pl.ds(jb * bk, bk), :], p,
                (((0,), (0,)), ((), ())),
                preferred_element_type=jnp.float32)
            acc_ref[gg] = acc_ref[gg] * alpha + pv
            return m_new, l_new

        def upd_diag(gg, m_prev, l_prev):
            cmat = (jax.lax.broadcasted_iota(jnp.int32, (bk, bq), 1)
                    - jax.lax.broadcasted_iota(jnp.int32, (bk, bq), 0))
            tri = cmat >= (jb * bk - i * bq)
            return update(gg, jnp.where(tri, s_ref[gg], _NEG), m_prev, l_prev)

        new_ms, new_ls = [], []
        for gg in range(g):
            m_new, l_new = jax.lax.cond(
                jb == jmax,
                functools.partial(upd_diag, gg, ms[gg], ls[gg]),
                lambda gg=gg, m=ms[gg], l=ls[gg]: update(gg, s_ref[gg], m, l))
            new_ms.append(m_new)
            new_ls.append(l_new)
        return tuple(new_ms), tuple(new_ls)

    for gg in range(g):
        acc_ref[gg] = jnp.zeros((d, bq), jnp.float32)
    m0 = tuple(jnp.full((1, bq), _NEG, jnp.float32) for _ in range(g))
    l0 = tuple(jnp.zeros((1, bq), jnp.float32) for _ in range(g))
    ms, ls = jax.lax.fori_loop(kmin, jmax + 1, body, (m0, l0))
    for gg in range(g):
        inv = 1.0 / ls[gg]
        o_ref[gg] = (acc_ref[gg] * inv).T


def _prep_body(q_ref, k_ref, v_ref, qo_ref, ko_ref, vo_ref, *, scale):
    qo_ref[...] = (q_ref[...] * scale).astype(jnp.bfloat16).transpose(1, 0, 2)
    ko_ref[...] = k_ref[...].astype(jnp.bfloat16).transpose(1, 0, 2)
    vo_ref[...] = v_ref[...].astype(jnp.bfloat16).transpose(1, 0, 2)


def _prep(q, k, v, scale):
    total, num_heads, d = q.shape
    tc = 512
    nchunks = total // tc
    spec_in = pl.BlockSpec((tc, num_heads, d), lambda c: (c, 0, 0))
    spec_out = pl.BlockSpec((num_heads, tc, d), lambda c: (0, c, 0))
    shp = jax.ShapeDtypeStruct((num_heads, total, d), jnp.bfloat16)
    return pl.pallas_call(
        functools.partial(_prep_body, scale=scale),
        grid=(nchunks,),
        in_specs=[spec_in, spec_in, spec_in],
        out_specs=[spec_out, spec_out, spec_out],
        out_shape=[shp, shp, shp],
        compiler_params=pltpu.CompilerParams(
            dimension_semantics=("arbitrary",),
        ),
    )(q, k, v)


def _post_body(x_ref, o_ref):
    o_ref[...] = x_ref[...].transpose(1, 0, 2)


def _post(x):
    num_heads, total, d = x.shape
    tc = 512
    nchunks = total // tc
    return pl.pallas_call(
        _post_body,
        grid=(nchunks,),
        in_specs=[pl.BlockSpec((num_heads, tc, d), lambda c: (0, c, 0))],
        out_specs=pl.BlockSpec((tc, num_heads, d), lambda c: (c, 0, 0)),
        out_shape=jax.ShapeDtypeStruct((total, num_heads, d), jnp.float32),
        compiler_params=pltpu.CompilerParams(
            dimension_semantics=("arbitrary",),
        ),
    )(x)


def kernel(q, k, v, cu_seqlens_q, cu_seqlens_k):
    total, num_heads, d = q.shape
    num_segs = cu_seqlens_q.shape[0] - 1
    scale = 1.0 / np.sqrt(d)
    assert _BK % _BQ == 0 and total % _BK == 0 and num_heads % _G == 0
    num_q = total // _BQ
    num_hg = num_heads // _G

    qs, ks, vs = _prep(q, k, v, scale * float(np.log2(np.e)))

    qblk = jnp.arange(num_q, dtype=jnp.int32) * _BQ
    seg_first = jnp.searchsorted(cu_seqlens_q, qblk, side="right") - 1
    seg_last = jnp.searchsorted(cu_seqlens_q, qblk + (_BQ - 1), side="right") - 1
    kmin_blk = (cu_seqlens_q[seg_first] // _BK).astype(jnp.int32)
    smax_blk = cu_seqlens_q[seg_last].astype(jnp.int32)

    body = functools.partial(_flash_body, num_segs=num_segs, g=_G,
                             bq=_BQ, bk=_BK)
    grid_spec = pltpu.PrefetchScalarGridSpec(
        num_scalar_prefetch=3,
        grid=(num_hg, num_q),
        in_specs=[
            pl.BlockSpec((_G, _BQ, d), lambda h, i, *_: (h, i, 0)),
            pl.BlockSpec((_G, total, d), lambda h, i, *_: (h, 0, 0)),
            pl.BlockSpec((_G, total, d), lambda h, i, *_: (h, 0, 0)),
        ],
        out_specs=pl.BlockSpec((_G, _BQ, d), lambda h, i, *_: (h, i, 0)),
        scratch_shapes=[
            pltpu.VMEM((_G, _BK, _BQ), jnp.float32),
            pltpu.VMEM((_G, d, _BQ), jnp.float32),
        ],
    )
    out_t = pl.pallas_call(
        body,
        grid_spec=grid_spec,
        out_shape=jax.ShapeDtypeStruct((num_heads, total, d), jnp.float32),
        compiler_params=pltpu.CompilerParams(
            dimension_semantics=("arbitrary", "arbitrary"),
        ),
    )(kmin_blk, smax_blk, cu_seqlens_q, qs, ks, vs)
    return _post(out_t)


# speedup vs baseline: 1.3698x; 1.3698x over previous
import functools

import jax
import jax.numpy as jnp
import numpy as np
from jax.experimental import pallas as pl
from jax.experimental.pallas import tpu as pltpu

_BQ = 512
_BK = 512
_G = 4
_NEG = -1e30


def _flash_body(kmin_ref, smax_ref, cu_ref, q_ref, k_ref, v_ref, o_ref,
                s_ref, acc_ref, *, num_segs, g, bq, bk):
    hg = pl.program_id(0)
    del hg
    i = pl.program_id(1)
    d = q_ref.shape[-1]

    kmin = kmin_ref[i]
    smax = smax_ref[i]
    jmax = (i * bq + bq - 1) // bk

    ones_bk = jnp.ones((bk, 8), jnp.bfloat16)

    def body(jb, carry):
        ms, ls = carry
        for gg in range(g):
            s_ref[gg] = jax.lax.dot_general(
                k_ref[gg, pl.ds(jb * bk, bk), :], q_ref[gg],
                (((1,), (1,)), ((), ())),
                preferred_element_type=jnp.float32)

        @pl.when(jb * bk < smax)
        def _segmask():
            qpos = i * bq + jax.lax.broadcasted_iota(jnp.int32, (1, bq), 1)
            seg_start = jnp.zeros((1, bq), jnp.int32)
            for b in range(1, num_segs + 1):
                c = cu_ref[b]
                seg_start = jnp.where(qpos >= c, c, seg_start)
            kpos = jb * bk + jax.lax.broadcasted_iota(jnp.int32, (bk, 1), 0)
            sel = kpos >= seg_start
            for gg in range(g):
                s_ref[gg] = jnp.where(sel, s_ref[gg], _NEG)

        def update(gg, s, m_prev, l_prev):
            m_cur = jnp.max(s, axis=0, keepdims=True)
            m_new = jnp.maximum(m_prev, m_cur)
            alpha = jnp.exp2(m_prev - m_new)
            p = jnp.exp2(s - m_new).astype(jnp.bfloat16)
            lsum = jax.lax.dot_general(
                ones_bk, p, (((0,), (0,)), ((), ())),
                preferred_element_type=jnp.float32)
            l_new = l_prev * alpha + lsum[0:1, :]
            pv = jax.lax.dot_general(
                v_ref[gg, pl.ds(jb * bk, bk), :], p,
                (((0,), (0,)), ((), ())),
                preferred_element_type=jnp.float32)
            acc_ref[gg] = acc_ref[gg] * alpha + pv
            return m_new, l_new

        def upd_diag(gg, m_prev, l_prev):
            tri = (jax.lax.broadcasted_iota(jnp.int32, (bk, bq), 1)
                   >= jax.lax.broadcasted_iota(jnp.int32, (bk, bq), 0))
            return update(gg, jnp.where(tri, s_ref[gg], _NEG), m_prev, l_prev)

        new_ms, new_ls = [], []
        for gg in range(g):
            m_new, l_new = jax.lax.cond(
                jb == jmax,
                functools.partial(upd_diag, gg, ms[gg], ls[gg]),
                lambda gg=gg, m=ms[gg], l=ls[gg]: update(gg, s_ref[gg], m, l))
            new_ms.append(m_new)
            new_ls.append(l_new)
        return tuple(new_ms), tuple(new_ls)

    for gg in range(g):
        acc_ref[gg] = jnp.zeros((d, bq), jnp.float32)
    m0 = tuple(jnp.full((1, bq), _NEG, jnp.float32) for _ in range(g))
    l0 = tuple(jnp.zeros((1, bq), jnp.float32) for _ in range(g))
    ms, ls = jax.lax.fori_loop(kmin, jmax + 1, body, (m0, l0))
    for gg in range(g):
        inv = 1.0 / ls[gg]
        o_ref[gg] = (acc_ref[gg] * inv).T


def _prep_body(q_ref, k_ref, v_ref, qo_ref, ko_ref, vo_ref, *, scale):
    qo_ref[...] = (q_ref[...] * scale).astype(jnp.bfloat16).transpose(1, 0, 2)
    ko_ref[...] = k_ref[...].astype(jnp.bfloat16).transpose(1, 0, 2)
    vo_ref[...] = v_ref[...].astype(jnp.bfloat16).transpose(1, 0, 2)


def _prep(q, k, v, scale):
    total, num_heads, d = q.shape
    tc = 512
    nchunks = total // tc
    spec_in = pl.BlockSpec((tc, num_heads, d), lambda c: (c, 0, 0))
    spec_out = pl.BlockSpec((num_heads, tc, d), lambda c: (0, c, 0))
    shp = jax.ShapeDtypeStruct((num_heads, total, d), jnp.bfloat16)
    return pl.pallas_call(
        functools.partial(_prep_body, scale=scale),
        grid=(nchunks,),
        in_specs=[spec_in, spec_in, spec_in],
        out_specs=[spec_out, spec_out, spec_out],
        out_shape=[shp, shp, shp],
        compiler_params=pltpu.CompilerParams(
            dimension_semantics=("arbitrary",),
        ),
    )(q, k, v)


def _post_body(x_ref, o_ref):
    o_ref[...] = x_ref[...].transpose(1, 0, 2)


def _post(x):
    num_heads, total, d = x.shape
    tc = 512
    nchunks = total // tc
    return pl.pallas_call(
        _post_body,
        grid=(nchunks,),
        in_specs=[pl.BlockSpec((num_heads, tc, d), lambda c: (0, c, 0))],
        out_specs=pl.BlockSpec((tc, num_heads, d), lambda c: (c, 0, 0)),
        out_shape=jax.ShapeDtypeStruct((total, num_heads, d), jnp.float32),
        compiler_params=pltpu.CompilerParams(
            dimension_semantics=("arbitrary",),
        ),
    )(x)


def kernel(q, k, v, cu_seqlens_q, cu_seqlens_k):
    total, num_heads, d = q.shape
    num_segs = cu_seqlens_q.shape[0] - 1
    scale = 1.0 / np.sqrt(d)
    assert _BQ == _BK and total % _BQ == 0 and num_heads % _G == 0
    num_q = total // _BQ
    num_hg = num_heads // _G

    qs, ks, vs = _prep(q, k, v, scale * float(np.log2(np.e)))

    qblk = jnp.arange(num_q, dtype=jnp.int32) * _BQ
    seg_first = jnp.searchsorted(cu_seqlens_q, qblk, side="right") - 1
    seg_last = jnp.searchsorted(cu_seqlens_q, qblk + (_BQ - 1), side="right") - 1
    kmin_blk = (cu_seqlens_q[seg_first] // _BK).astype(jnp.int32)
    smax_blk = cu_seqlens_q[seg_last].astype(jnp.int32)

    body = functools.partial(_flash_body, num_segs=num_segs, g=_G,
                             bq=_BQ, bk=_BK)
    grid_spec = pltpu.PrefetchScalarGridSpec(
        num_scalar_prefetch=3,
        grid=(num_hg, num_q),
        in_specs=[
            pl.BlockSpec((_G, _BQ, d), lambda h, i, *_: (h, i, 0)),
            pl.BlockSpec((_G, total, d), lambda h, i, *_: (h, 0, 0)),
            pl.BlockSpec((_G, total, d), lambda h, i, *_: (h, 0, 0)),
        ],
        out_specs=pl.BlockSpec((_G, _BQ, d), lambda h, i, *_: (h, i, 0)),
        scratch_shapes=[
            pltpu.VMEM((_G, _BK, _BQ), jnp.float32),
            pltpu.VMEM((_G, d, _BQ), jnp.float32),
        ],
    )
    out_t = pl.pallas_call(
        body,
        grid_spec=grid_spec,
        out_shape=jax.ShapeDtypeStruct((num_heads, total, d), jnp.float32),
        compiler_params=pltpu.CompilerParams(
            dimension_semantics=("arbitrary", "arbitrary"),
        ),
    )(kmin_blk, smax_blk, cu_seqlens_q, qs, ks, vs)
    return _post(out_t)
